# Initial kernel scaffold; baseline (speedup 1.0000x reference)
import jax
import jax.numpy as jnp
from jax.experimental import pallas as pl


def kernel(x, edge_index, edge_type, batch, W1, root1, b1, W2, root2, b2, W3, root3, b3, Wm1, bm1, Wm2, bm2):
    raise NotImplementedError("write your pallas kernel here")


# speedup vs baseline: 8.7341x; 8.7341x over previous
import functools

import jax
import jax.numpy as jnp
from jax import lax
from jax.experimental import pallas as pl
from jax.experimental.pallas import tpu as pltpu
from jax.experimental.pallas import tpu_sc as plsc

N = 10000
E = 320000
R = 7
DIN = 128
NHID = 256
NOUT = 128
G = 128

NC = 2
NS = 16
L = 16
NW = NC * NS

B = 80
HF = NHID // 2

EPC = E // NS
EPW = E // NW
NPT = N // NS
CNT_PAD = 81920

_i32 = jnp.int32
_f32 = jnp.float32


def _mesh():
  return plsc.VectorSubcoreMesh(
      core_axis_name="c", subcore_axis_name="s",
      num_cores=NC, num_subcores=NS)


def _iota16():
  return lax.iota(_i32, L)


def _sc_weights(src, dst, edge_type):
  @functools.partial(
      pl.kernel,
      out_type=jax.ShapeDtypeStruct((E,), _f32),
      mesh=_mesh(),
      compiler_params=pltpu.CompilerParams(needs_layout_passes=False),
      scratch_types=[
          pltpu.VMEM_SHARED((CNT_PAD,), _f32),
          pltpu.VMEM((2560,), _f32),
          pltpu.VMEM((B,), _i32),
          pltpu.VMEM((B,), _i32),
          pltpu.VMEM((B,), _i32),
          pltpu.VMEM((B,), _f32),
          pltpu.VMEM((CNT_PAD - 1920,), _f32),
          pltpu.VMEM((B,), _f32),
      ],
  )
  def k(src_hbm, dst_hbm, et_hbm, w_hbm,
        cnt_sp, zb, dstb, tb, idxb, ones, invb, wb):
    c = lax.axis_index("c")
    s = lax.axis_index("s")
    wid = s * NC + c

    def zinit(i, _):
      zb[pl.ds(i * L, L)] = jnp.zeros((L,), _f32)
      return 0
    lax.fori_loop(0, 160, zinit, 0)
    for i in range(B // L):
      ones[pl.ds(i * L, L)] = jnp.ones((L,), _f32)

    pltpu.sync_copy(zb, cnt_sp.at[pl.ds(s * 5120, 2560)])
    pltpu.sync_copy(zb, cnt_sp.at[pl.ds(s * 5120 + 2560, 2560)])
    plsc.subcore_barrier()

    def count_body(bi, _):
      off = s * EPC + bi * B
      pltpu.sync_copy(dst_hbm.at[pl.ds(off, B)], dstb)
      pltpu.sync_copy(et_hbm.at[pl.ds(off, B)], tb)
      for kk in range(B // L):
        dv = dstb[pl.ds(kk * L, L)]
        tv = tb[pl.ds(kk * L, L)]
        idxb[pl.ds(kk * L, L)] = dv * 8 + tv
      pltpu.sync_copy(ones, cnt_sp.at[idxb], add=True)
      return 0
    lax.fori_loop(0, EPC // B, count_body, 0)
    plsc.subcore_barrier()

    pltpu.sync_copy(cnt_sp.at[pl.ds(0, CNT_PAD - 1920)], invb)
    def inv_body(i, _):
      v = invb[pl.ds(i * L, L)]
      invb[pl.ds(i * L, L)] = 1.0 / jnp.maximum(v, 1.0)
      return 0
    lax.fori_loop(0, (CNT_PAD - 1920) // L, inv_body, 0)

    def w_body(bi, _):
      off = wid * EPW + bi * B
      pltpu.sync_copy(dst_hbm.at[pl.ds(off, B)], dstb)
      pltpu.sync_copy(et_hbm.at[pl.ds(off, B)], tb)
      for kk in range(B // L):
        dv = dstb[pl.ds(kk * L, L)]
        tv = tb[pl.ds(kk * L, L)]
        wb[pl.ds(kk * L, L)] = plsc.load_gather(invb, [dv * 8 + tv])
      pltpu.sync_copy(wb, w_hbm.at[pl.ds(off, B)])
      return 0
    lax.fori_loop(0, EPW // B, w_body, 0)

  return k(src, dst, edge_type)


def _sc_layer(table, src, dst, edge_type, w):
  @functools.partial(
      pl.kernel,
      out_type=jax.ShapeDtypeStruct((2 * N, HF), _f32),
      mesh=_mesh(),
      compiler_params=pltpu.CompilerParams(needs_layout_passes=False),
      scratch_types=[
          pltpu.VMEM_SHARED((N, HF), _f32),
          pltpu.VMEM((16, HF), _f32),
          pltpu.VMEM((B,), _i32),
          pltpu.VMEM((B,), _i32),
          pltpu.VMEM((B,), _i32),
          pltpu.VMEM((B,), _i32),
          pltpu.VMEM((B,), _f32),
          pltpu.VMEM((B, HF), _f32),
          pltpu.VMEM((16, HF), _f32),
          pltpu.SemaphoreType.DMA,
      ],
  )
  def k(tab_hbm, src_hbm, dst_hbm, et_hbm, w_hbm, out_hbm,
        acc_sp, zb, srcb, dstb, tb, gidxb, wb, rows, ob, sem):
    c = lax.axis_index("c")
    s = lax.axis_index("s")

    for i in range(16):
      for j in range(HF // L):
        zb[i, pl.ds(j * L, L)] = jnp.zeros((L,), _f32)
    def z_body(i, _):
      blk = s + i * NS
      @pl.when(blk < N // 16)
      def _():
        pltpu.sync_copy(zb, acc_sp.at[pl.ds(blk * 16, 16)])
      return 0
    lax.fori_loop(0, (N // 16 + NS - 1) // NS, z_body, 0)
    plsc.subcore_barrier()

    def body(bi, _):
      off = s * EPC + bi * B
      pltpu.sync_copy(src_hbm.at[pl.ds(off, B)], srcb)
      pltpu.sync_copy(dst_hbm.at[pl.ds(off, B)], dstb)
      pltpu.sync_copy(et_hbm.at[pl.ds(off, B)], tb)
      pltpu.sync_copy(w_hbm.at[pl.ds(off, B)], wb)
      base = c * (R * N)
      for kk in range(B // L):
        sv = srcb[pl.ds(kk * L, L)]
        tv = tb[pl.ds(kk * L, L)]
        gidxb[pl.ds(kk * L, L)] = tv * N + sv + base
      pltpu.async_copy(tab_hbm.at[gidxb], rows, sem).wait()
      iot = _iota16()
      for kk in range(B // L):
        wv = wb[pl.ds(kk * L, L)]
        for m in range(L):
          sc = lax.reduce_sum(jnp.where(iot == m, wv, 0.0), axes=(0,))
          e = kk * L + m
          for j in range(HF // L):
            rows[e, pl.ds(j * L, L)] = rows[e, pl.ds(j * L, L)] * sc
      pltpu.sync_copy(rows, acc_sp.at[dstb], add=True)
      return 0
    lax.fori_loop(0, EPC // B, body, 0)
    plsc.subcore_barrier()

    def wb_body(i, _):
      blk = s + i * NS
      @pl.when(blk < N // 16)
      def _():
        pltpu.sync_copy(acc_sp.at[pl.ds(blk * 16, 16)], ob)
        pltpu.sync_copy(ob, out_hbm.at[pl.ds(c * N + blk * 16, 16)])
      return 0
    lax.fori_loop(0, (N // 16 + NS - 1) // NS, wb_body, 0)

  return k(table, src, dst, edge_type, w)


def _sc_pool(h3p, batch_p):
  NPAD = 10240
  NPW = NPAD // NW

  @functools.partial(
      pl.kernel,
      out_type=jax.ShapeDtypeStruct((NW * G * NHID,), _f32),
      mesh=_mesh(),
      compiler_params=pltpu.CompilerParams(needs_layout_passes=False),
      scratch_types=[
          pltpu.VMEM((G * NHID,), _f32),
          pltpu.VMEM((L, NHID), _f32),
          pltpu.VMEM((L,), _i32),
      ],
  )
  def k(h_hbm, b_hbm, out_hbm, gacc, rowsb, batchb):
    c = lax.axis_index("c")
    s = lax.axis_index("s")
    wid = s * NC + c

    def z_body(i, _):
      gacc[pl.ds(i * L, L)] = jnp.zeros((L,), _f32)
      return 0
    lax.fori_loop(0, G * NHID // L, z_body, 0)

    iot = _iota16()

    def chunk_body(kk, _):
      off = wid * NPW + kk * L
      pltpu.sync_copy(h_hbm.at[pl.ds(off, L)], rowsb)
      pltpu.sync_copy(b_hbm.at[pl.ds(off, L)], batchb)
      bv = batchb[pl.ds(0, L)]
      for m in range(L):
        gid = lax.reduce_max(jnp.where(iot == m, bv, 0), axes=(0,))
        base = gid * NHID
        for j in range(NHID // L):
          idxv = base + (j * L + iot)
          cur = plsc.load_gather(gacc, [idxv])
          nv = jnp.maximum(cur, rowsb[m, pl.ds(j * L, L)])
          plsc.store_scatter(gacc, [idxv], nv)
      return 0
    lax.fori_loop(0, NPW // L, chunk_body, 0)

    pltpu.sync_copy(gacc, out_hbm.at[pl.ds(wid * (G * NHID), G * NHID)])

  return k(h3p, batch_p)


BN = 400
NB = N // BN


def _tc_head(x, W, root, b):
  def body(x_ref, w_ref, root_ref, b_ref, h2_ref, r1_ref):
    r = pl.program_id(1)
    xb = x_ref[...]
    h = jnp.dot(xb, w_ref[0], preferred_element_type=_f32)
    h2_ref[0, 0] = h[:, :HF]
    h2_ref[1, 0] = h[:, HF:]
    @pl.when(r == 0)
    def _():
      r1_ref[...] = (jnp.dot(xb, root_ref[...], preferred_element_type=_f32)
                     + b_ref[...])

  d = x.shape[1]
  return pl.pallas_call(
      body,
      grid=(NB, R),
      in_specs=[
          pl.BlockSpec((BN, d), lambda i, r: (i, 0)),
          pl.BlockSpec((1, d, NHID), lambda i, r: (r, 0, 0)),
          pl.BlockSpec((d, NHID), lambda i, r: (0, 0)),
          pl.BlockSpec((1, NHID), lambda i, r: (0, 0)),
      ],
      out_specs=[
          pl.BlockSpec((2, 1, BN, HF), lambda i, r: (0, r, i, 0)),
          pl.BlockSpec((BN, NHID), lambda i, r: (i, 0)),
      ],
      out_shape=[
          jax.ShapeDtypeStruct((2, R, N, HF), _f32),
          jax.ShapeDtypeStruct((N, NHID), _f32),
      ],
  )(x, W, root, b.reshape(1, NHID))


def _tc_mid(Rprev, msg, W, root, b):
  def body(rp_ref, m_ref, w_ref, root_ref, b_ref, h2_ref, rn_ref):
    r = pl.program_id(1)
    h = jax.nn.relu(rp_ref[...] +
                    jnp.concatenate([m_ref[0], m_ref[1]], axis=1))
    hh = jnp.dot(h, w_ref[0], preferred_element_type=_f32)
    h2_ref[0, 0] = hh[:, :HF]
    h2_ref[1, 0] = hh[:, HF:]
    @pl.when(r == 0)
    def _():
      rn_ref[...] = (jnp.dot(h, root_ref[...], preferred_element_type=_f32)
                     + b_ref[...])

  return pl.pallas_call(
      body,
      grid=(NB, R),
      in_specs=[
          pl.BlockSpec((BN, NHID), lambda i, r: (i, 0)),
          pl.BlockSpec((2, BN, HF), lambda i, r: (0, i, 0)),
          pl.BlockSpec((1, NHID, NHID), lambda i, r: (r, 0, 0)),
          pl.BlockSpec((NHID, NHID), lambda i, r: (0, 0)),
          pl.BlockSpec((1, NHID), lambda i, r: (0, 0)),
      ],
      out_specs=[
          pl.BlockSpec((2, 1, BN, HF), lambda i, r: (0, r, i, 0)),
          pl.BlockSpec((BN, NHID), lambda i, r: (i, 0)),
      ],
      out_shape=[
          jax.ShapeDtypeStruct((2, R, N, HF), _f32),
          jax.ShapeDtypeStruct((N, NHID), _f32),
      ],
  )(Rprev, msg, W, root, b.reshape(1, NHID))


def _tc_relu(Rprev, msg):
  def body(rp_ref, m_ref, o_ref):
    o_ref[...] = jax.nn.relu(rp_ref[...] +
                             jnp.concatenate([m_ref[0], m_ref[1]], axis=1))

  return pl.pallas_call(
      body,
      grid=(NB,),
      in_specs=[
          pl.BlockSpec((BN, NHID), lambda i: (i, 0)),
          pl.BlockSpec((2, BN, HF), lambda i: (0, i, 0)),
      ],
      out_specs=pl.BlockSpec((BN, NHID), lambda i: (i, 0)),
      out_shape=jax.ShapeDtypeStruct((N, NHID), _f32),
  )(Rprev, msg)


def _tc_pool_mlp(parts, Wm1, bm1, Wm2, bm2):
  def body(p_ref, w1_ref, b1_ref, w2_ref, b2_ref, o_ref):
    g = jnp.max(p_ref[...], axis=0)
    gg = jax.nn.relu(jnp.dot(g, w1_ref[...], preferred_element_type=_f32)
                     + b1_ref[...])
    o_ref[...] = (jnp.dot(gg, w2_ref[...], preferred_element_type=_f32)
                  + b2_ref[...])

  return pl.pallas_call(
      body,
      out_shape=jax.ShapeDtypeStruct((G, NOUT), _f32),
  )(parts, Wm1, bm1.reshape(1, NHID), Wm2, bm2.reshape(1, NOUT))


def kernel(x, edge_index, edge_type, batch,
           W1, root1, b1, W2, root2, b2, W3, root3, b3,
           Wm1, bm1, Wm2, bm2):
  src = edge_index[0]
  dst = edge_index[1]
  w = _sc_weights(src, dst, edge_type)

  H2, R1 = _tc_head(x, W1, root1, b1)
  msg1 = _sc_layer(H2.reshape(2 * R * N, HF), src, dst, edge_type, w)

  H2, R2 = _tc_mid(R1, msg1.reshape(2, N, HF), W2, root2, b2)
  msg2 = _sc_layer(H2.reshape(2 * R * N, HF), src, dst, edge_type, w)

  H2, R3 = _tc_mid(R2, msg2.reshape(2, N, HF), W3, root3, b3)
  msg3 = _sc_layer(H2.reshape(2 * R * N, HF), src, dst, edge_type, w)

  h3 = _tc_relu(R3, msg3.reshape(2, N, HF))

  h3p = jnp.concatenate([h3, jnp.zeros((10240 - N, NHID), _f32)], axis=0)
  batch_p = jnp.concatenate([batch, jnp.zeros((10240 - N,), _i32)], axis=0)
  parts = _sc_pool(h3p, batch_p)

  return _tc_pool_mlp(parts.reshape(NW, G, NHID), Wm1, bm1, Wm2, bm2)


# speedup vs baseline: 12.2424x; 1.4017x over previous
import functools

import jax
import jax.numpy as jnp
from jax import lax
from jax.experimental import pallas as pl
from jax.experimental.pallas import tpu as pltpu
from jax.experimental.pallas import tpu_sc as plsc

N = 10000
E = 320000
R = 7
DIN = 128
NHID = 256
NOUT = 128
G = 128

NC = 2
NS = 16
L = 16
NW = NC * NS

B = 80
HF = NHID // 2

EPC = E // NS
EPW = E // NW
NPT = N // NS
CNT_PAD = 81920

_i32 = jnp.int32
_f32 = jnp.float32


def _mesh():
  return plsc.VectorSubcoreMesh(
      core_axis_name="c", subcore_axis_name="s",
      num_cores=NC, num_subcores=NS)


def _iota16():
  return lax.iota(_i32, L)


CH = 2000


def _sc_weights(src, dst, edge_type):
  @functools.partial(
      pl.kernel,
      out_type=[jax.ShapeDtypeStruct((E * L,), _f32),
                jax.ShapeDtypeStruct((E,), _i32)],
      mesh=_mesh(),
      compiler_params=pltpu.CompilerParams(needs_layout_passes=False),
      scratch_types=[
          pltpu.VMEM_SHARED((CNT_PAD,), _f32),
          pltpu.VMEM((2560,), _f32),
          pltpu.VMEM((CH,), _i32),
          pltpu.VMEM((CH,), _i32),
          pltpu.VMEM((CH,), _i32),
          pltpu.VMEM((B,), _i32),
          pltpu.VMEM((B,), _f32),
          pltpu.VMEM((CNT_PAD - 1920,), _f32),
          pltpu.VMEM((B * L,), _f32),
          pltpu.VMEM((CH,), _i32),
      ],
  )
  def k(src_hbm, dst_hbm, et_hbm, w16_hbm, gidx_hbm,
        cnt_sp, zb, srcc, dstc, tc, idxb, ones, invb, wb16, gc):
    c = lax.axis_index("c")
    s = lax.axis_index("s")
    wid = s * NC + c
    iot = _iota16()

    def zinit(i, _):
      zb[pl.ds(i * L, L)] = jnp.zeros((L,), _f32)
      return 0
    lax.fori_loop(0, 160, zinit, 0)
    for i in range(B // L):
      ones[pl.ds(i * L, L)] = jnp.ones((L,), _f32)

    pltpu.sync_copy(zb, cnt_sp.at[pl.ds(s * 5120, 2560)])
    pltpu.sync_copy(zb, cnt_sp.at[pl.ds(s * 5120 + 2560, 2560)])
    plsc.subcore_barrier()

    def count_chunk(ci, _):
      coff = s * EPC + ci * CH
      pltpu.sync_copy(dst_hbm.at[pl.ds(coff, CH)], dstc)
      pltpu.sync_copy(et_hbm.at[pl.ds(coff, CH)], tc)
      def count_body(bi, _):
        for kk in range(B // L):
          dv = dstc[pl.ds(bi * B + kk * L, L)]
          tv = tc[pl.ds(bi * B + kk * L, L)]
          idxb[pl.ds(kk * L, L)] = dv * 8 + tv
        pltpu.sync_copy(ones, cnt_sp.at[idxb], add=True)
        return 0
      lax.fori_loop(0, CH // B, count_body, 0)
      return 0
    lax.fori_loop(0, EPC // CH, count_chunk, 0)
    plsc.subcore_barrier()

    pltpu.sync_copy(cnt_sp.at[pl.ds(0, CNT_PAD - 1920)], invb)
    def inv_body(i, _):
      v = invb[pl.ds(i * L, L)]
      invb[pl.ds(i * L, L)] = 1.0 / jnp.maximum(v, 1.0)
      return 0
    lax.fori_loop(0, (CNT_PAD - 1920) // L, inv_body, 0)

    def w_chunk(ci, _):
      coff = wid * EPW + ci * CH
      pltpu.sync_copy(src_hbm.at[pl.ds(coff, CH)], srcc)
      pltpu.sync_copy(dst_hbm.at[pl.ds(coff, CH)], dstc)
      pltpu.sync_copy(et_hbm.at[pl.ds(coff, CH)], tc)
      def gi_body(i, _):
        sv = srcc[pl.ds(i * L, L)]
        tv = tc[pl.ds(i * L, L)]
        gc[pl.ds(i * L, L)] = tv * N + sv
        return 0
      lax.fori_loop(0, CH // L, gi_body, 0)
      pltpu.sync_copy(gc, gidx_hbm.at[pl.ds(coff, CH)])
      def w_body(bi, _):
        for kk in range(B // L):
          dv = dstc[pl.ds(bi * B + kk * L, L)]
          tv = tc[pl.ds(bi * B + kk * L, L)]
          wv = plsc.load_gather(invb, [dv * 8 + tv])
          for j in range(L):
            plsc.store_scatter(wb16, [kk * (L * L) + iot * L + j], wv)
        pltpu.sync_copy(
            wb16, w16_hbm.at[pl.ds((coff + bi * B) * L, B * L)])
        return 0
      lax.fori_loop(0, CH // B, w_body, 0)
      return 0
    lax.fori_loop(0, EPW // CH, w_chunk, 0)

  return k(src, dst, edge_type)


def _sc_layer(table, dst, gidx, w16):
  @functools.partial(
      pl.kernel,
      out_type=jax.ShapeDtypeStruct((2 * N, HF), _f32),
      mesh=_mesh(),
      compiler_params=pltpu.CompilerParams(needs_layout_passes=False),
      scratch_types=[
          pltpu.VMEM_SHARED((N, HF), _f32),
          pltpu.VMEM((16, HF), _f32),
          pltpu.VMEM((CH,), _i32),
          pltpu.VMEM((CH,), _i32),
          pltpu.VMEM((B,), _i32),
          pltpu.VMEM((B,), _i32),
          pltpu.VMEM((B * L,), _f32),
          pltpu.VMEM((B, HF), _f32),
          pltpu.VMEM((16, HF), _f32),
          pltpu.SemaphoreType.DMA,
      ],
  )
  def k(tab_hbm, dst_hbm, gidx_hbm, w16_hbm, out_hbm,
        acc_sp, zb, dstc, gc, gidxb, dstb, wb16, rows, ob, sem):
    c = lax.axis_index("c")
    s = lax.axis_index("s")

    for i in range(16):
      for j in range(HF // L):
        zb[i, pl.ds(j * L, L)] = jnp.zeros((L,), _f32)
    def z_body(i, _):
      blk = s + i * NS
      @pl.when(blk < N // 16)
      def _():
        pltpu.sync_copy(zb, acc_sp.at[pl.ds(blk * 16, 16)])
      return 0
    lax.fori_loop(0, (N // 16 + NS - 1) // NS, z_body, 0)
    plsc.subcore_barrier()

    base = c * (R * N)
    def chunk_body(ci, _):
      coff = s * EPC + ci * CH
      pltpu.sync_copy(dst_hbm.at[pl.ds(coff, CH)], dstc)
      pltpu.sync_copy(gidx_hbm.at[pl.ds(coff, CH)], gc)
      def body(bi, _):
        pltpu.sync_copy(
            w16_hbm.at[pl.ds((coff + bi * B) * L, B * L)], wb16)
        for kk in range(B // L):
          gidxb[pl.ds(kk * L, L)] = gc[pl.ds(bi * B + kk * L, L)] + base
          dstb[pl.ds(kk * L, L)] = dstc[pl.ds(bi * B + kk * L, L)]
        pltpu.async_copy(tab_hbm.at[gidxb], rows, sem).wait()
        for e in range(B):
          wv = wb16[pl.ds(e * L, L)]
          for j in range(HF // L):
            rows[e, pl.ds(j * L, L)] = rows[e, pl.ds(j * L, L)] * wv
        pltpu.sync_copy(rows, acc_sp.at[dstb], add=True)
        return 0
      lax.fori_loop(0, CH // B, body, 0)
      return 0
    lax.fori_loop(0, EPC // CH, chunk_body, 0)
    plsc.subcore_barrier()

    def wb_body(i, _):
      blk = s + i * NS
      @pl.when(blk < N // 16)
      def _():
        pltpu.sync_copy(acc_sp.at[pl.ds(blk * 16, 16)], ob)
        pltpu.sync_copy(ob, out_hbm.at[pl.ds(c * N + blk * 16, 16)])
      return 0
    lax.fori_loop(0, (N // 16 + NS - 1) // NS, wb_body, 0)

  return k(table, dst, gidx, w16)


def _sc_pool(h3p, batch_p):
  NPAD = 10240
  NPW = NPAD // NW

  @functools.partial(
      pl.kernel,
      out_type=jax.ShapeDtypeStruct((NW * G * NHID,), _f32),
      mesh=_mesh(),
      compiler_params=pltpu.CompilerParams(needs_layout_passes=False),
      scratch_types=[
          pltpu.VMEM((G * NHID,), _f32),
          pltpu.VMEM((L, NHID), _f32),
          pltpu.VMEM((L,), _i32),
      ],
  )
  def k(h_hbm, b_hbm, out_hbm, gacc, rowsb, batchb):
    c = lax.axis_index("c")
    s = lax.axis_index("s")
    wid = s * NC + c

    def z_body(i, _):
      gacc[pl.ds(i * L, L)] = jnp.zeros((L,), _f32)
      return 0
    lax.fori_loop(0, G * NHID // L, z_body, 0)

    iot = _iota16()

    def chunk_body(kk, _):
      off = wid * NPW + kk * L
      pltpu.sync_copy(h_hbm.at[pl.ds(off, L)], rowsb)
      pltpu.sync_copy(b_hbm.at[pl.ds(off, L)], batchb)
      bv = batchb[pl.ds(0, L)]
      for m in range(L):
        gid = lax.reduce_max(jnp.where(iot == m, bv, 0), axes=(0,))
        base = gid * NHID
        for j in range(NHID // L):
          idxv = base + (j * L + iot)
          cur = plsc.load_gather(gacc, [idxv])
          nv = jnp.maximum(cur, rowsb[m, pl.ds(j * L, L)])
          plsc.store_scatter(gacc, [idxv], nv)
      return 0
    lax.fori_loop(0, NPW // L, chunk_body, 0)

    pltpu.sync_copy(gacc, out_hbm.at[pl.ds(wid * (G * NHID), G * NHID)])

  return k(h3p, batch_p)


BN = 400
NB = N // BN


def _tc_head(x, W, root, b):
  def body(x_ref, w_ref, root_ref, b_ref, h2_ref, r1_ref):
    r = pl.program_id(1)
    xb = x_ref[...]
    h = jnp.dot(xb, w_ref[0], preferred_element_type=_f32)
    h2_ref[0, 0] = h[:, :HF]
    h2_ref[1, 0] = h[:, HF:]
    @pl.when(r == 0)
    def _():
      r1_ref[...] = (jnp.dot(xb, root_ref[...], preferred_element_type=_f32)
                     + b_ref[...])

  d = x.shape[1]
  return pl.pallas_call(
      body,
      grid=(NB, R),
      in_specs=[
          pl.BlockSpec((BN, d), lambda i, r: (i, 0)),
          pl.BlockSpec((1, d, NHID), lambda i, r: (r, 0, 0)),
          pl.BlockSpec((d, NHID), lambda i, r: (0, 0)),
          pl.BlockSpec((1, NHID), lambda i, r: (0, 0)),
      ],
      out_specs=[
          pl.BlockSpec((2, 1, BN, HF), lambda i, r: (0, r, i, 0)),
          pl.BlockSpec((BN, NHID), lambda i, r: (i, 0)),
      ],
      out_shape=[
          jax.ShapeDtypeStruct((2, R, N, HF), _f32),
          jax.ShapeDtypeStruct((N, NHID), _f32),
      ],
  )(x, W, root, b.reshape(1, NHID))


def _tc_mid(Rprev, msg, W, root, b):
  def body(rp_ref, m_ref, w_ref, root_ref, b_ref, h2_ref, rn_ref):
    r = pl.program_id(1)
    h = jax.nn.relu(rp_ref[...] +
                    jnp.concatenate([m_ref[0], m_ref[1]], axis=1))
    hh = jnp.dot(h, w_ref[0], preferred_element_type=_f32)
    h2_ref[0, 0] = hh[:, :HF]
    h2_ref[1, 0] = hh[:, HF:]
    @pl.when(r == 0)
    def _():
      rn_ref[...] = (jnp.dot(h, root_ref[...], preferred_element_type=_f32)
                     + b_ref[...])

  return pl.pallas_call(
      body,
      grid=(NB, R),
      in_specs=[
          pl.BlockSpec((BN, NHID), lambda i, r: (i, 0)),
          pl.BlockSpec((2, BN, HF), lambda i, r: (0, i, 0)),
          pl.BlockSpec((1, NHID, NHID), lambda i, r: (r, 0, 0)),
          pl.BlockSpec((NHID, NHID), lambda i, r: (0, 0)),
          pl.BlockSpec((1, NHID), lambda i, r: (0, 0)),
      ],
      out_specs=[
          pl.BlockSpec((2, 1, BN, HF), lambda i, r: (0, r, i, 0)),
          pl.BlockSpec((BN, NHID), lambda i, r: (i, 0)),
      ],
      out_shape=[
          jax.ShapeDtypeStruct((2, R, N, HF), _f32),
          jax.ShapeDtypeStruct((N, NHID), _f32),
      ],
  )(Rprev, msg, W, root, b.reshape(1, NHID))


def _tc_relu(Rprev, msg):
  def body(rp_ref, m_ref, o_ref):
    o_ref[...] = jax.nn.relu(rp_ref[...] +
                             jnp.concatenate([m_ref[0], m_ref[1]], axis=1))

  return pl.pallas_call(
      body,
      grid=(NB,),
      in_specs=[
          pl.BlockSpec((BN, NHID), lambda i: (i, 0)),
          pl.BlockSpec((2, BN, HF), lambda i: (0, i, 0)),
      ],
      out_specs=pl.BlockSpec((BN, NHID), lambda i: (i, 0)),
      out_shape=jax.ShapeDtypeStruct((N, NHID), _f32),
  )(Rprev, msg)


def _tc_pool_mlp(parts, Wm1, bm1, Wm2, bm2):
  def body(p_ref, w1_ref, b1_ref, w2_ref, b2_ref, o_ref):
    g = jnp.max(p_ref[...], axis=0)
    gg = jax.nn.relu(jnp.dot(g, w1_ref[...], preferred_element_type=_f32)
                     + b1_ref[...])
    o_ref[...] = (jnp.dot(gg, w2_ref[...], preferred_element_type=_f32)
                  + b2_ref[...])

  return pl.pallas_call(
      body,
      out_shape=jax.ShapeDtypeStruct((G, NOUT), _f32),
  )(parts, Wm1, bm1.reshape(1, NHID), Wm2, bm2.reshape(1, NOUT))


def kernel(x, edge_index, edge_type, batch,
           W1, root1, b1, W2, root2, b2, W3, root3, b3,
           Wm1, bm1, Wm2, bm2):
  src = edge_index[0]
  dst = edge_index[1]
  w16, gidx = _sc_weights(src, dst, edge_type)

  H2, R1 = _tc_head(x, W1, root1, b1)
  msg1 = _sc_layer(H2.reshape(2 * R * N, HF), dst, gidx, w16)

  H2, R2 = _tc_mid(R1, msg1.reshape(2, N, HF), W2, root2, b2)
  msg2 = _sc_layer(H2.reshape(2 * R * N, HF), dst, gidx, w16)

  H2, R3 = _tc_mid(R2, msg2.reshape(2, N, HF), W3, root3, b3)
  msg3 = _sc_layer(H2.reshape(2 * R * N, HF), dst, gidx, w16)

  h3 = _tc_relu(R3, msg3.reshape(2, N, HF))

  h3p = jnp.concatenate([h3, jnp.zeros((10240 - N, NHID), _f32)], axis=0)
  batch_p = jnp.concatenate([batch, jnp.zeros((10240 - N,), _i32)], axis=0)
  parts = _sc_pool(h3p, batch_p)

  return _tc_pool_mlp(parts.reshape(NW, G, NHID), Wm1, bm1, Wm2, bm2)


# speedup vs baseline: 19.4549x; 1.5891x over previous
import functools

import jax
import jax.numpy as jnp
from jax import lax
from jax.experimental import pallas as pl
from jax.experimental.pallas import tpu as pltpu
from jax.experimental.pallas import tpu_sc as plsc

N = 10000
E = 320000
R = 7
DIN = 128
NHID = 256
NOUT = 128
G = 128

NC = 2
NS = 16
L = 16
NW = NC * NS

B = 80
HF = NHID // 2

EPC = E // NS
EPW = E // NW
NPT = N // NS
CNT_PAD = 81920

_i32 = jnp.int32
_f32 = jnp.float32


def _mesh():
  return plsc.VectorSubcoreMesh(
      core_axis_name="c", subcore_axis_name="s",
      num_cores=NC, num_subcores=NS)


def _iota16():
  return lax.iota(_i32, L)


CH = 2000


def _sc_weights(src, dst, edge_type):
  @functools.partial(
      pl.kernel,
      out_type=[jax.ShapeDtypeStruct((E * L,), _f32),
                jax.ShapeDtypeStruct((E,), _i32)],
      mesh=_mesh(),
      compiler_params=pltpu.CompilerParams(needs_layout_passes=False),
      scratch_types=[
          pltpu.VMEM_SHARED((CNT_PAD,), _f32),
          pltpu.VMEM((2560,), _f32),
          pltpu.VMEM((CH,), _i32),
          pltpu.VMEM((CH,), _i32),
          pltpu.VMEM((CH,), _i32),
          pltpu.VMEM((B,), _i32),
          pltpu.VMEM((B,), _f32),
          pltpu.VMEM((CNT_PAD - 1920,), _f32),
          pltpu.VMEM((B * L,), _f32),
          pltpu.VMEM((CH,), _i32),
      ],
  )
  def k(src_hbm, dst_hbm, et_hbm, w16_hbm, gidx_hbm,
        cnt_sp, zb, srcc, dstc, tc, idxb, ones, invb, wb16, gc):
    c = lax.axis_index("c")
    s = lax.axis_index("s")
    wid = s * NC + c
    iot = _iota16()

    def zinit(i, _):
      zb[pl.ds(i * L, L)] = jnp.zeros((L,), _f32)
      return 0
    lax.fori_loop(0, 160, zinit, 0)
    for i in range(B // L):
      ones[pl.ds(i * L, L)] = jnp.ones((L,), _f32)

    pltpu.sync_copy(zb, cnt_sp.at[pl.ds(s * 5120, 2560)])
    pltpu.sync_copy(zb, cnt_sp.at[pl.ds(s * 5120 + 2560, 2560)])
    plsc.subcore_barrier()

    def count_chunk(ci, _):
      coff = s * EPC + ci * CH
      pltpu.sync_copy(dst_hbm.at[pl.ds(coff, CH)], dstc)
      pltpu.sync_copy(et_hbm.at[pl.ds(coff, CH)], tc)
      def count_body(bi, _):
        for kk in range(B // L):
          dv = dstc[pl.ds(bi * B + kk * L, L)]
          tv = tc[pl.ds(bi * B + kk * L, L)]
          idxb[pl.ds(kk * L, L)] = dv * 8 + tv
        pltpu.sync_copy(ones, cnt_sp.at[idxb], add=True)
        return 0
      lax.fori_loop(0, CH // B, count_body, 0)
      return 0
    lax.fori_loop(0, EPC // CH, count_chunk, 0)
    plsc.subcore_barrier()

    pltpu.sync_copy(cnt_sp.at[pl.ds(0, CNT_PAD - 1920)], invb)
    def inv_body(i, _):
      v = invb[pl.ds(i * L, L)]
      invb[pl.ds(i * L, L)] = 1.0 / jnp.maximum(v, 1.0)
      return 0
    lax.fori_loop(0, (CNT_PAD - 1920) // L, inv_body, 0)

    def w_chunk(ci, _):
      coff = wid * EPW + ci * CH
      pltpu.sync_copy(src_hbm.at[pl.ds(coff, CH)], srcc)
      pltpu.sync_copy(dst_hbm.at[pl.ds(coff, CH)], dstc)
      pltpu.sync_copy(et_hbm.at[pl.ds(coff, CH)], tc)
      def gi_body(i, _):
        sv = srcc[pl.ds(i * L, L)]
        tv = tc[pl.ds(i * L, L)]
        gc[pl.ds(i * L, L)] = tv * N + sv
        return 0
      lax.fori_loop(0, CH // L, gi_body, 0)
      pltpu.sync_copy(gc, gidx_hbm.at[pl.ds(coff, CH)])
      def w_body(bi, _):
        for kk in range(B // L):
          dv = dstc[pl.ds(bi * B + kk * L, L)]
          tv = tc[pl.ds(bi * B + kk * L, L)]
          wv = plsc.load_gather(invb, [dv * 8 + tv])
          for j in range(L):
            plsc.store_scatter(wb16, [kk * (L * L) + iot * L + j], wv)
        pltpu.sync_copy(
            wb16, w16_hbm.at[pl.ds((coff + bi * B) * L, B * L)])
        return 0
      lax.fori_loop(0, CH // B, w_body, 0)
      return 0
    lax.fori_loop(0, EPW // CH, w_chunk, 0)

  return k(src, dst, edge_type)


def _sc_layer(tabA, tabB, dst, gidx, w16):
  NBAT = EPC // B

  @functools.partial(
      pl.kernel,
      out_type=jax.ShapeDtypeStruct((2 * N, HF), _f32),
      mesh=_mesh(),
      compiler_params=pltpu.CompilerParams(needs_layout_passes=False),
      scratch_types=[
          pltpu.VMEM_SHARED((N, HF), _f32),
          pltpu.VMEM((16, HF), _f32),
          [pltpu.VMEM((B,), _i32) for _ in range(3)],
          [pltpu.VMEM((B,), _i32) for _ in range(3)],
          [pltpu.VMEM((B,), _i32) for _ in range(3)],
          [pltpu.VMEM((B * L,), _f32) for _ in range(3)],
          [pltpu.VMEM((B, HF), _f32) for _ in range(3)],
          pltpu.VMEM((16, HF), _f32),
          [pltpu.SemaphoreType.DMA for _ in range(3)],
          [pltpu.SemaphoreType.DMA for _ in range(3)],
          [pltpu.SemaphoreType.DMA for _ in range(3)],
      ],
  )
  def k(tabA_hbm, tabB_hbm, dst_hbm, gidx_hbm, w16_hbm, out_hbm,
        acc_sp, zb, gidxb, dstS, dstX, wb16, rows, ob,
        semG, semI, semC):
    c = lax.axis_index("c")
    s = lax.axis_index("s")

    for i in range(16):
      for j in range(HF // L):
        zb[i, pl.ds(j * L, L)] = jnp.zeros((L,), _f32)
    def z_body(i, _):
      blk = s + i * NS
      @pl.when(blk < N // 16)
      def _():
        pltpu.sync_copy(zb, acc_sp.at[pl.ds(blk * 16, 16)])
      return 0
    lax.fori_loop(0, (N // 16 + NS - 1) // NS, z_body, 0)
    plsc.subcore_barrier()

    def start_idx(slot, b):
      off = s * EPC + b * B
      pltpu.async_copy(dst_hbm.at[pl.ds(off, B)], dstS[slot], semI[slot])
      pltpu.async_copy(gidx_hbm.at[pl.ds(off, B)], gidxb[slot], semI[slot])

    def wait_idx(slot):
      pltpu.make_async_copy(dst_hbm.at[pl.ds(0, B)], dstS[slot],
                            semI[slot]).wait()
      pltpu.make_async_copy(gidx_hbm.at[pl.ds(0, B)], gidxb[slot],
                            semI[slot]).wait()

    def start_gw(slot, b):
      @pl.when(c == 0)
      def _():
        pltpu.async_copy(tabA_hbm.at[gidxb[slot]], rows[slot], semG[slot])
      @pl.when(c == 1)
      def _():
        pltpu.async_copy(tabB_hbm.at[gidxb[slot]], rows[slot], semG[slot])
      pltpu.async_copy(
          w16_hbm.at[pl.ds((s * EPC + b * B) * L, B * L)], wb16[slot],
          semG[slot])

    def wait_gw(slot):
      pltpu.make_async_copy(tabA_hbm.at[gidxb[slot]], rows[slot],
                            semG[slot]).wait()
      pltpu.make_async_copy(
          w16_hbm.at[pl.ds(0, B * L)], wb16[slot], semG[slot]).wait()

    def scale(slot):
      for e in range(B):
        wv = wb16[slot][pl.ds(e * L, L)]
        for j in range(HF // L):
          rows[slot][e, pl.ds(j * L, L)] = (
              rows[slot][e, pl.ds(j * L, L)] * wv)

    def start_scatter(slot):
      for kk in range(B // L):
        dstX[slot][pl.ds(kk * L, L)] = dstS[slot][pl.ds(kk * L, L)]
      pltpu.async_copy(rows[slot], acc_sp.at[dstX[slot]], semC[slot],
                       add=True)

    def wait_c(slot):
      pltpu.make_async_copy(tabA_hbm.at[gidxb[slot]], rows[slot],
                            semC[slot]).wait()

    for b in range(3):
      start_idx(b, b)
    for b in range(2):
      wait_idx(b)
      start_gw(b, b)

    def steady(p, _):
      b0 = p * 3
      for h in range(3):
        q, q2 = h, (h + 2) % 3
        wait_gw(q)
        scale(q)
        start_scatter(q)
        if h == 0:
          @pl.when(p > 0)
          def _():
            wait_c(q2)
        else:
          wait_c(q2)
        wait_idx(q2)
        start_gw(q2, b0 + h + 2)
        start_idx(q, b0 + h + 3)
      return 0
    lax.fori_loop(0, (NBAT - 4) // 3, steady, 0)

    for b in range(NBAT - 4, NBAT):
      q = b % 3
      wait_gw(q)
      scale(q)
      start_scatter(q)
      if b + 2 < NBAT:
        q2 = (b + 2) % 3
        wait_c(q2)
        wait_idx(q2)
        start_gw(q2, b + 2)
      if b + 3 < NBAT:
        start_idx(q, b + 3)
    wait_c((NBAT - 3) % 3)
    wait_c((NBAT - 2) % 3)
    wait_c((NBAT - 1) % 3)
    plsc.subcore_barrier()

    def wb_body(i, _):
      blk = s + i * NS
      @pl.when(blk < N // 16)
      def _():
        pltpu.sync_copy(acc_sp.at[pl.ds(blk * 16, 16)], ob)
        pltpu.sync_copy(ob, out_hbm.at[pl.ds(c * N + blk * 16, 16)])
      return 0
    lax.fori_loop(0, (N // 16 + NS - 1) // NS, wb_body, 0)

  return k(tabA, tabB, dst, gidx, w16)


def _sc_pool(h3p, batch_p):
  NPAD = 10240
  NPW = NPAD // NW

  @functools.partial(
      pl.kernel,
      out_type=jax.ShapeDtypeStruct((NW * G * NHID,), _f32),
      mesh=_mesh(),
      compiler_params=pltpu.CompilerParams(needs_layout_passes=False),
      scratch_types=[
          pltpu.VMEM((G * NHID,), _f32),
          pltpu.VMEM((L, NHID), _f32),
          pltpu.VMEM((L,), _i32),
      ],
  )
  def k(h_hbm, b_hbm, out_hbm, gacc, rowsb, batchb):
    c = lax.axis_index("c")
    s = lax.axis_index("s")
    wid = s * NC + c

    def z_body(i, _):
      gacc[pl.ds(i * L, L)] = jnp.zeros((L,), _f32)
      return 0
    lax.fori_loop(0, G * NHID // L, z_body, 0)

    iot = _iota16()

    def chunk_body(kk, _):
      off = wid * NPW + kk * L
      pltpu.sync_copy(h_hbm.at[pl.ds(off, L)], rowsb)
      pltpu.sync_copy(b_hbm.at[pl.ds(off, L)], batchb)
      bv = batchb[pl.ds(0, L)]
      for m in range(L):
        gid = lax.reduce_max(jnp.where(iot == m, bv, 0), axes=(0,))
        base = gid * NHID
        for j in range(NHID // L):
          idxv = base + (j * L + iot)
          cur = plsc.load_gather(gacc, [idxv])
          nv = jnp.maximum(cur, rowsb[m, pl.ds(j * L, L)])
          plsc.store_scatter(gacc, [idxv], nv)
      return 0
    lax.fori_loop(0, NPW // L, chunk_body, 0)

    pltpu.sync_copy(gacc, out_hbm.at[pl.ds(wid * (G * NHID), G * NHID)])

  return k(h3p, batch_p)


BN = 400
NB = N // BN


def _tc_head(x, W, root, b):
  def body(x_ref, w_ref, root_ref, b_ref, h2a_ref, h2b_ref, r1_ref):
    r = pl.program_id(1)
    xb = x_ref[...]
    h = jnp.dot(xb, w_ref[0], preferred_element_type=_f32)
    h2a_ref[0] = h[:, :HF]
    h2b_ref[0] = h[:, HF:]
    @pl.when(r == 0)
    def _():
      r1_ref[...] = (jnp.dot(xb, root_ref[...], preferred_element_type=_f32)
                     + b_ref[...])

  d = x.shape[1]
  return pl.pallas_call(
      body,
      grid=(NB, R),
      in_specs=[
          pl.BlockSpec((BN, d), lambda i, r: (i, 0)),
          pl.BlockSpec((1, d, NHID), lambda i, r: (r, 0, 0)),
          pl.BlockSpec((d, NHID), lambda i, r: (0, 0)),
          pl.BlockSpec((1, NHID), lambda i, r: (0, 0)),
      ],
      out_specs=[
          pl.BlockSpec((1, BN, HF), lambda i, r: (r, i, 0)),
          pl.BlockSpec((1, BN, HF), lambda i, r: (r, i, 0)),
          pl.BlockSpec((BN, NHID), lambda i, r: (i, 0)),
      ],
      out_shape=[
          jax.ShapeDtypeStruct((R, N, HF), _f32),
          jax.ShapeDtypeStruct((R, N, HF), _f32),
          jax.ShapeDtypeStruct((N, NHID), _f32),
      ],
  )(x, W, root, b.reshape(1, NHID))


def _tc_mid(Rprev, msg, W, root, b):
  def body(rp_ref, m_ref, w_ref, root_ref, b_ref, h2a_ref, h2b_ref, rn_ref):
    r = pl.program_id(1)
    h = jax.nn.relu(rp_ref[...] +
                    jnp.concatenate([m_ref[0], m_ref[1]], axis=1))
    hh = jnp.dot(h, w_ref[0], preferred_element_type=_f32)
    h2a_ref[0] = hh[:, :HF]
    h2b_ref[0] = hh[:, HF:]
    @pl.when(r == 0)
    def _():
      rn_ref[...] = (jnp.dot(h, root_ref[...], preferred_element_type=_f32)
                     + b_ref[...])

  return pl.pallas_call(
      body,
      grid=(NB, R),
      in_specs=[
          pl.BlockSpec((BN, NHID), lambda i, r: (i, 0)),
          pl.BlockSpec((2, BN, HF), lambda i, r: (0, i, 0)),
          pl.BlockSpec((1, NHID, NHID), lambda i, r: (r, 0, 0)),
          pl.BlockSpec((NHID, NHID), lambda i, r: (0, 0)),
          pl.BlockSpec((1, NHID), lambda i, r: (0, 0)),
      ],
      out_specs=[
          pl.BlockSpec((1, BN, HF), lambda i, r: (r, i, 0)),
          pl.BlockSpec((1, BN, HF), lambda i, r: (r, i, 0)),
          pl.BlockSpec((BN, NHID), lambda i, r: (i, 0)),
      ],
      out_shape=[
          jax.ShapeDtypeStruct((R, N, HF), _f32),
          jax.ShapeDtypeStruct((R, N, HF), _f32),
          jax.ShapeDtypeStruct((N, NHID), _f32),
      ],
  )(Rprev, msg, W, root, b.reshape(1, NHID))


def _tc_relu(Rprev, msg):
  def body(rp_ref, m_ref, o_ref):
    o_ref[...] = jax.nn.relu(rp_ref[...] +
                             jnp.concatenate([m_ref[0], m_ref[1]], axis=1))

  return pl.pallas_call(
      body,
      grid=(NB,),
      in_specs=[
          pl.BlockSpec((BN, NHID), lambda i: (i, 0)),
          pl.BlockSpec((2, BN, HF), lambda i: (0, i, 0)),
      ],
      out_specs=pl.BlockSpec((BN, NHID), lambda i: (i, 0)),
      out_shape=jax.ShapeDtypeStruct((N, NHID), _f32),
  )(Rprev, msg)


def _tc_pool_mlp(parts, Wm1, bm1, Wm2, bm2):
  def body(p_ref, w1_ref, b1_ref, w2_ref, b2_ref, o_ref):
    g = jnp.max(p_ref[...], axis=0)
    gg = jax.nn.relu(jnp.dot(g, w1_ref[...], preferred_element_type=_f32)
                     + b1_ref[...])
    o_ref[...] = (jnp.dot(gg, w2_ref[...], preferred_element_type=_f32)
                  + b2_ref[...])

  return pl.pallas_call(
      body,
      out_shape=jax.ShapeDtypeStruct((G, NOUT), _f32),
  )(parts, Wm1, bm1.reshape(1, NHID), Wm2, bm2.reshape(1, NOUT))


def kernel(x, edge_index, edge_type, batch,
           W1, root1, b1, W2, root2, b2, W3, root3, b3,
           Wm1, bm1, Wm2, bm2):
  src = edge_index[0]
  dst = edge_index[1]
  w16, gidx = _sc_weights(src, dst, edge_type)

  Ha, Hb, R1 = _tc_head(x, W1, root1, b1)
  msg1 = _sc_layer(Ha.reshape(R * N, HF), Hb.reshape(R * N, HF),
                   dst, gidx, w16)

  Ha, Hb, R2 = _tc_mid(R1, msg1.reshape(2, N, HF), W2, root2, b2)
  msg2 = _sc_layer(Ha.reshape(R * N, HF), Hb.reshape(R * N, HF),
                   dst, gidx, w16)

  Ha, Hb, R3 = _tc_mid(R2, msg2.reshape(2, N, HF), W3, root3, b3)
  msg3 = _sc_layer(Ha.reshape(R * N, HF), Hb.reshape(R * N, HF),
                   dst, gidx, w16)

  h3 = _tc_relu(R3, msg3.reshape(2, N, HF))

  h3p = jnp.concatenate([h3, jnp.zeros((10240 - N, NHID), _f32)], axis=0)
  batch_p = jnp.concatenate([batch, jnp.zeros((10240 - N,), _i32)], axis=0)
  parts = _sc_pool(h3p, batch_p)

  return _tc_pool_mlp(parts.reshape(NW, G, NHID), Wm1, bm1, Wm2, bm2)


# speedup vs baseline: 19.7756x; 1.0165x over previous
import functools

import jax
import jax.numpy as jnp
from jax import lax
from jax.experimental import pallas as pl
from jax.experimental.pallas import tpu as pltpu
from jax.experimental.pallas import tpu_sc as plsc

N = 10000
E = 320000
R = 7
DIN = 128
NHID = 256
NOUT = 128
G = 128

NC = 2
NS = 16
L = 16
NW = NC * NS

B = 80
HF = NHID // 2

EPC = E // NS
EPW = E // NW
NPT = N // NS
CNT_PAD = 81920

_i32 = jnp.int32
_f32 = jnp.float32


def _mesh():
  return plsc.VectorSubcoreMesh(
      core_axis_name="c", subcore_axis_name="s",
      num_cores=NC, num_subcores=NS)


def _iota16():
  return lax.iota(_i32, L)


CH = 2000


def _sc_weights(src, dst, edge_type):
  @functools.partial(
      pl.kernel,
      out_type=[jax.ShapeDtypeStruct((E * L,), _f32),
                jax.ShapeDtypeStruct((E,), _i32)],
      mesh=_mesh(),
      compiler_params=pltpu.CompilerParams(needs_layout_passes=False),
      scratch_types=[
          pltpu.VMEM_SHARED((CNT_PAD,), _f32),
          pltpu.VMEM((2560,), _f32),
          pltpu.VMEM((CH,), _i32),
          pltpu.VMEM((CH,), _i32),
          pltpu.VMEM((CH,), _i32),
          pltpu.VMEM((B,), _i32),
          pltpu.VMEM((B,), _f32),
          pltpu.VMEM((CNT_PAD - 1920,), _f32),
          pltpu.VMEM((B * L,), _f32),
          pltpu.VMEM((CH,), _i32),
      ],
  )
  def k(src_hbm, dst_hbm, et_hbm, w16_hbm, gidx_hbm,
        cnt_sp, zb, srcc, dstc, tc, idxb, ones, invb, wb16, gc):
    c = lax.axis_index("c")
    s = lax.axis_index("s")
    wid = s * NC + c
    iot = _iota16()

    def zinit(i, _):
      zb[pl.ds(i * L, L)] = jnp.zeros((L,), _f32)
      return 0
    lax.fori_loop(0, 160, zinit, 0)
    for i in range(B // L):
      ones[pl.ds(i * L, L)] = jnp.ones((L,), _f32)

    pltpu.sync_copy(zb, cnt_sp.at[pl.ds(s * 5120, 2560)])
    pltpu.sync_copy(zb, cnt_sp.at[pl.ds(s * 5120 + 2560, 2560)])
    plsc.subcore_barrier()

    def count_chunk(ci, _):
      coff = s * EPC + ci * CH
      pltpu.sync_copy(dst_hbm.at[pl.ds(coff, CH)], dstc)
      pltpu.sync_copy(et_hbm.at[pl.ds(coff, CH)], tc)
      def count_body(bi, _):
        for kk in range(B // L):
          dv = dstc[pl.ds(bi * B + kk * L, L)]
          tv = tc[pl.ds(bi * B + kk * L, L)]
          idxb[pl.ds(kk * L, L)] = dv * 8 + tv
        pltpu.sync_copy(ones, cnt_sp.at[idxb], add=True)
        return 0
      lax.fori_loop(0, CH // B, count_body, 0)
      return 0
    lax.fori_loop(0, EPC // CH, count_chunk, 0)
    plsc.subcore_barrier()

    pltpu.sync_copy(cnt_sp.at[pl.ds(0, CNT_PAD - 1920)], invb)
    def inv_body(i, _):
      v = invb[pl.ds(i * L, L)]
      invb[pl.ds(i * L, L)] = 1.0 / jnp.maximum(v, 1.0)
      return 0
    lax.fori_loop(0, (CNT_PAD - 1920) // L, inv_body, 0)

    def w_chunk(ci, _):
      coff = wid * EPW + ci * CH
      pltpu.sync_copy(src_hbm.at[pl.ds(coff, CH)], srcc)
      pltpu.sync_copy(dst_hbm.at[pl.ds(coff, CH)], dstc)
      pltpu.sync_copy(et_hbm.at[pl.ds(coff, CH)], tc)
      def gi_body(i, _):
        sv = srcc[pl.ds(i * L, L)]
        tv = tc[pl.ds(i * L, L)]
        gc[pl.ds(i * L, L)] = tv * N + sv
        return 0
      lax.fori_loop(0, CH // L, gi_body, 0)
      pltpu.sync_copy(gc, gidx_hbm.at[pl.ds(coff, CH)])
      def w_body(bi, _):
        for kk in range(B // L):
          dv = dstc[pl.ds(bi * B + kk * L, L)]
          tv = tc[pl.ds(bi * B + kk * L, L)]
          wv = plsc.load_gather(invb, [dv * 8 + tv])
          for j in range(L):
            plsc.store_scatter(wb16, [kk * (L * L) + iot * L + j], wv)
        pltpu.sync_copy(
            wb16, w16_hbm.at[pl.ds((coff + bi * B) * L, B * L)])
        return 0
      lax.fori_loop(0, CH // B, w_body, 0)
      return 0
    lax.fori_loop(0, EPW // CH, w_chunk, 0)

  return k(src, dst, edge_type)


def _sc_layer(tabA, tabB, dst, gidx, w16):
  NBAT = EPC // B

  @functools.partial(
      pl.kernel,
      out_type=jax.ShapeDtypeStruct((2 * N, HF), _f32),
      mesh=_mesh(),
      compiler_params=pltpu.CompilerParams(needs_layout_passes=False),
      scratch_types=[
          pltpu.VMEM_SHARED((N, HF), _f32),
          pltpu.VMEM((16, HF), _f32),
          [pltpu.VMEM((B,), _i32) for _ in range(3)],
          [pltpu.VMEM((B,), _i32) for _ in range(3)],
          [pltpu.VMEM((B,), _i32) for _ in range(3)],
          [pltpu.VMEM((B * L,), _f32) for _ in range(3)],
          [pltpu.VMEM((B, HF), _f32) for _ in range(3)],
          pltpu.VMEM((16, HF), _f32),
          [pltpu.SemaphoreType.DMA for _ in range(3)],
          [pltpu.SemaphoreType.DMA for _ in range(3)],
          [pltpu.SemaphoreType.DMA for _ in range(3)],
          pltpu.SemaphoreType.DMA,
      ],
  )
  def k(tabA_hbm, tabB_hbm, dst_hbm, gidx_hbm, w16_hbm, out_hbm,
        acc_sp, zb, gidxb, dstS, dstX, wb16, rows, ob,
        semG, semI, semC, semZ):
    c = lax.axis_index("c")
    s = lax.axis_index("s")

    for i in range(16):
      for j in range(HF // L):
        zb[i, pl.ds(j * L, L)] = jnp.zeros((L,), _f32)
    def z_body(i, _):
      blk = s + i * NS
      @pl.when(blk < N // 16)
      def _():
        pltpu.async_copy(zb, acc_sp.at[pl.ds(blk * 16, 16)], semZ)
      return 0
    lax.fori_loop(0, (N // 16 + NS - 1) // NS, z_body, 0)
    def z_drain(i, _):
      blk = s + i * NS
      @pl.when(blk < N // 16)
      def _():
        pltpu.make_async_copy(zb, acc_sp.at[pl.ds(blk * 16, 16)],
                              semZ).wait()
      return 0
    lax.fori_loop(0, (N // 16 + NS - 1) // NS, z_drain, 0)
    plsc.subcore_barrier()

    def start_idx(slot, b):
      off = s * EPC + b * B
      pltpu.async_copy(dst_hbm.at[pl.ds(off, B)], dstS[slot], semI[slot])
      pltpu.async_copy(gidx_hbm.at[pl.ds(off, B)], gidxb[slot], semI[slot])

    def wait_idx(slot):
      pltpu.make_async_copy(dst_hbm.at[pl.ds(0, B)], dstS[slot],
                            semI[slot]).wait()
      pltpu.make_async_copy(gidx_hbm.at[pl.ds(0, B)], gidxb[slot],
                            semI[slot]).wait()

    def start_gw(slot, b):
      @pl.when(c == 0)
      def _():
        pltpu.async_copy(tabA_hbm.at[gidxb[slot]], rows[slot], semG[slot])
      @pl.when(c == 1)
      def _():
        pltpu.async_copy(tabB_hbm.at[gidxb[slot]], rows[slot], semG[slot])
      pltpu.async_copy(
          w16_hbm.at[pl.ds((s * EPC + b * B) * L, B * L)], wb16[slot],
          semG[slot])

    def wait_gw(slot):
      pltpu.make_async_copy(tabA_hbm.at[gidxb[slot]], rows[slot],
                            semG[slot]).wait()
      pltpu.make_async_copy(
          w16_hbm.at[pl.ds(0, B * L)], wb16[slot], semG[slot]).wait()

    def scale(slot):
      for e in range(B):
        wv = wb16[slot][pl.ds(e * L, L)]
        for j in range(HF // L):
          rows[slot][e, pl.ds(j * L, L)] = (
              rows[slot][e, pl.ds(j * L, L)] * wv)

    def start_scatter(slot):
      for kk in range(B // L):
        dstX[slot][pl.ds(kk * L, L)] = dstS[slot][pl.ds(kk * L, L)]
      pltpu.async_copy(rows[slot], acc_sp.at[dstX[slot]], semC[slot],
                       add=True)

    def wait_c(slot):
      pltpu.make_async_copy(tabA_hbm.at[gidxb[slot]], rows[slot],
                            semC[slot]).wait()

    for b in range(3):
      start_idx(b, b)
    for b in range(2):
      wait_idx(b)
      start_gw(b, b)

    def steady(p, _):
      b0 = p * 3
      for h in range(3):
        q, q2 = h, (h + 2) % 3
        wait_gw(q)
        scale(q)
        start_scatter(q)
        if h == 0:
          @pl.when(p > 0)
          def _():
            wait_c(q2)
        else:
          wait_c(q2)
        wait_idx(q2)
        start_gw(q2, b0 + h + 2)
        start_idx(q, b0 + h + 3)
      return 0
    lax.fori_loop(0, (NBAT - 4) // 3, steady, 0)

    for b in range(NBAT - 4, NBAT):
      q = b % 3
      wait_gw(q)
      scale(q)
      start_scatter(q)
      if b + 2 < NBAT:
        q2 = (b + 2) % 3
        wait_c(q2)
        wait_idx(q2)
        start_gw(q2, b + 2)
      if b + 3 < NBAT:
        start_idx(q, b + 3)
    wait_c((NBAT - 3) % 3)
    wait_c((NBAT - 2) % 3)
    wait_c((NBAT - 1) % 3)
    plsc.subcore_barrier()

    def wb_body(i, _):
      blk = s + i * NS
      @pl.when(blk < N // B)
      def _():
        pltpu.sync_copy(acc_sp.at[pl.ds(blk * B, B)], rows[0])
        pltpu.sync_copy(rows[0], out_hbm.at[pl.ds(c * N + blk * B, B)])
      return 0
    lax.fori_loop(0, (N // B + NS - 1) // NS, wb_body, 0)

  return k(tabA, tabB, dst, gidx, w16)


def _sc_pool(h3p, batch_p):
  NPAD = 10240
  NPW = NPAD // NW

  @functools.partial(
      pl.kernel,
      out_type=jax.ShapeDtypeStruct((NW * G * NHID,), _f32),
      mesh=_mesh(),
      compiler_params=pltpu.CompilerParams(needs_layout_passes=False),
      scratch_types=[
          pltpu.VMEM((G * NHID,), _f32),
          pltpu.VMEM((L, NHID), _f32),
          pltpu.VMEM((L,), _i32),
      ],
  )
  def k(h_hbm, b_hbm, out_hbm, gacc, rowsb, batchb):
    c = lax.axis_index("c")
    s = lax.axis_index("s")
    wid = s * NC + c

    def z_body(i, _):
      gacc[pl.ds(i * L, L)] = jnp.zeros((L,), _f32)
      return 0
    lax.fori_loop(0, G * NHID // L, z_body, 0)

    iot = _iota16()

    def chunk_body(kk, _):
      off = wid * NPW + kk * L
      pltpu.sync_copy(h_hbm.at[pl.ds(off, L)], rowsb)
      pltpu.sync_copy(b_hbm.at[pl.ds(off, L)], batchb)
      bv = batchb[pl.ds(0, L)]
      for m in range(L):
        gid = lax.reduce_max(jnp.where(iot == m, bv, 0), axes=(0,))
        base = gid * NHID
        for j in range(NHID // L):
          idxv = base + (j * L + iot)
          cur = plsc.load_gather(gacc, [idxv])
          nv = jnp.maximum(cur, rowsb[m, pl.ds(j * L, L)])
          plsc.store_scatter(gacc, [idxv], nv)
      return 0
    lax.fori_loop(0, NPW // L, chunk_body, 0)

    pltpu.sync_copy(gacc, out_hbm.at[pl.ds(wid * (G * NHID), G * NHID)])

  return k(h3p, batch_p)


BN = 400
NB = N // BN


def _tc_head(x, W, root, b):
  def body(x_ref, w_ref, root_ref, b_ref, h2a_ref, h2b_ref, r1_ref):
    r = pl.program_id(1)
    xb = x_ref[...]
    h = jnp.dot(xb, w_ref[0], preferred_element_type=_f32)
    h2a_ref[0] = h[:, :HF]
    h2b_ref[0] = h[:, HF:]
    @pl.when(r == 0)
    def _():
      r1_ref[...] = (jnp.dot(xb, root_ref[...], preferred_element_type=_f32)
                     + b_ref[...])

  d = x.shape[1]
  return pl.pallas_call(
      body,
      grid=(NB, R),
      in_specs=[
          pl.BlockSpec((BN, d), lambda i, r: (i, 0)),
          pl.BlockSpec((1, d, NHID), lambda i, r: (r, 0, 0)),
          pl.BlockSpec((d, NHID), lambda i, r: (0, 0)),
          pl.BlockSpec((1, NHID), lambda i, r: (0, 0)),
      ],
      out_specs=[
          pl.BlockSpec((1, BN, HF), lambda i, r: (r, i, 0)),
          pl.BlockSpec((1, BN, HF), lambda i, r: (r, i, 0)),
          pl.BlockSpec((BN, NHID), lambda i, r: (i, 0)),
      ],
      out_shape=[
          jax.ShapeDtypeStruct((R, N, HF), _f32),
          jax.ShapeDtypeStruct((R, N, HF), _f32),
          jax.ShapeDtypeStruct((N, NHID), _f32),
      ],
  )(x, W, root, b.reshape(1, NHID))


def _tc_mid(Rprev, msg, W, root, b):
  def body(rp_ref, m_ref, w_ref, root_ref, b_ref, h2a_ref, h2b_ref, rn_ref):
    r = pl.program_id(1)
    h = jax.nn.relu(rp_ref[...] +
                    jnp.concatenate([m_ref[0], m_ref[1]], axis=1))
    hh = jnp.dot(h, w_ref[0], preferred_element_type=_f32)
    h2a_ref[0] = hh[:, :HF]
    h2b_ref[0] = hh[:, HF:]
    @pl.when(r == 0)
    def _():
      rn_ref[...] = (jnp.dot(h, root_ref[...], preferred_element_type=_f32)
                     + b_ref[...])

  return pl.pallas_call(
      body,
      grid=(NB, R),
      in_specs=[
          pl.BlockSpec((BN, NHID), lambda i, r: (i, 0)),
          pl.BlockSpec((2, BN, HF), lambda i, r: (0, i, 0)),
          pl.BlockSpec((1, NHID, NHID), lambda i, r: (r, 0, 0)),
          pl.BlockSpec((NHID, NHID), lambda i, r: (0, 0)),
          pl.BlockSpec((1, NHID), lambda i, r: (0, 0)),
      ],
      out_specs=[
          pl.BlockSpec((1, BN, HF), lambda i, r: (r, i, 0)),
          pl.BlockSpec((1, BN, HF), lambda i, r: (r, i, 0)),
          pl.BlockSpec((BN, NHID), lambda i, r: (i, 0)),
      ],
      out_shape=[
          jax.ShapeDtypeStruct((R, N, HF), _f32),
          jax.ShapeDtypeStruct((R, N, HF), _f32),
          jax.ShapeDtypeStruct((N, NHID), _f32),
      ],
  )(Rprev, msg, W, root, b.reshape(1, NHID))


def _tc_relu(Rprev, msg):
  def body(rp_ref, m_ref, o_ref):
    o_ref[...] = jax.nn.relu(rp_ref[...] +
                             jnp.concatenate([m_ref[0], m_ref[1]], axis=1))

  return pl.pallas_call(
      body,
      grid=(NB,),
      in_specs=[
          pl.BlockSpec((BN, NHID), lambda i: (i, 0)),
          pl.BlockSpec((2, BN, HF), lambda i: (0, i, 0)),
      ],
      out_specs=pl.BlockSpec((BN, NHID), lambda i: (i, 0)),
      out_shape=jax.ShapeDtypeStruct((N, NHID), _f32),
  )(Rprev, msg)


def _tc_pool_mlp(parts, Wm1, bm1, Wm2, bm2):
  def body(p_ref, w1_ref, b1_ref, w2_ref, b2_ref, o_ref):
    g = jnp.max(p_ref[...], axis=0)
    gg = jax.nn.relu(jnp.dot(g, w1_ref[...], preferred_element_type=_f32)
                     + b1_ref[...])
    o_ref[...] = (jnp.dot(gg, w2_ref[...], preferred_element_type=_f32)
                  + b2_ref[...])

  return pl.pallas_call(
      body,
      out_shape=jax.ShapeDtypeStruct((G, NOUT), _f32),
  )(parts, Wm1, bm1.reshape(1, NHID), Wm2, bm2.reshape(1, NOUT))


def kernel(x, edge_index, edge_type, batch,
           W1, root1, b1, W2, root2, b2, W3, root3, b3,
           Wm1, bm1, Wm2, bm2):
  src = edge_index[0]
  dst = edge_index[1]
  w16, gidx = _sc_weights(src, dst, edge_type)

  Ha, Hb, R1 = _tc_head(x, W1, root1, b1)
  msg1 = _sc_layer(Ha.reshape(R * N, HF), Hb.reshape(R * N, HF),
                   dst, gidx, w16)

  Ha, Hb, R2 = _tc_mid(R1, msg1.reshape(2, N, HF), W2, root2, b2)
  msg2 = _sc_layer(Ha.reshape(R * N, HF), Hb.reshape(R * N, HF),
                   dst, gidx, w16)

  Ha, Hb, R3 = _tc_mid(R2, msg2.reshape(2, N, HF), W3, root3, b3)
  msg3 = _sc_layer(Ha.reshape(R * N, HF), Hb.reshape(R * N, HF),
                   dst, gidx, w16)

  h3 = _tc_relu(R3, msg3.reshape(2, N, HF))

  h3p = jnp.concatenate([h3, jnp.zeros((10240 - N, NHID), _f32)], axis=0)
  batch_p = jnp.concatenate([batch, jnp.zeros((10240 - N,), _i32)], axis=0)
  parts = _sc_pool(h3p, batch_p)

  return _tc_pool_mlp(parts.reshape(NW, G, NHID), Wm1, bm1, Wm2, bm2)


# speedup vs baseline: 19.8744x; 1.0050x over previous
import functools

import jax
import jax.numpy as jnp
from jax import lax
from jax.experimental import pallas as pl
from jax.experimental.pallas import tpu as pltpu
from jax.experimental.pallas import tpu_sc as plsc

N = 10000
E = 320000
R = 7
DIN = 128
NHID = 256
NOUT = 128
G = 128

NC = 2
NS = 16
L = 16
NW = NC * NS

B = 80
HF = NHID // 2

EPC = E // NS
EPW = E // NW
NPT = N // NS
CNT_PAD = 81920

_i32 = jnp.int32
_f32 = jnp.float32


def _mesh():
  return plsc.VectorSubcoreMesh(
      core_axis_name="c", subcore_axis_name="s",
      num_cores=NC, num_subcores=NS)


def _iota16():
  return lax.iota(_i32, L)


CH = 2000


def _sc_weights(src, dst, edge_type):
  @functools.partial(
      pl.kernel,
      out_type=[jax.ShapeDtypeStruct((E * L,), _f32),
                jax.ShapeDtypeStruct((E,), _i32)],
      mesh=_mesh(),
      compiler_params=pltpu.CompilerParams(needs_layout_passes=False),
      scratch_types=[
          pltpu.VMEM_SHARED((CNT_PAD,), _f32),
          pltpu.VMEM((2560,), _f32),
          pltpu.VMEM((CH,), _i32),
          pltpu.VMEM((CH,), _i32),
          pltpu.VMEM((CH,), _i32),
          pltpu.VMEM((B,), _i32),
          pltpu.VMEM((B,), _f32),
          pltpu.VMEM((CNT_PAD - 1920,), _f32),
          pltpu.VMEM((B * L,), _f32),
          pltpu.VMEM((CH,), _i32),
      ],
  )
  def k(src_hbm, dst_hbm, et_hbm, w16_hbm, gidx_hbm,
        cnt_sp, zb, srcc, dstc, tc, idxb, ones, invb, wb16, gc):
    c = lax.axis_index("c")
    s = lax.axis_index("s")
    wid = s * NC + c
    iot = _iota16()

    def zinit(i, _):
      zb[pl.ds(i * L, L)] = jnp.zeros((L,), _f32)
      return 0
    lax.fori_loop(0, 160, zinit, 0)
    for i in range(B // L):
      ones[pl.ds(i * L, L)] = jnp.ones((L,), _f32)

    pltpu.sync_copy(zb, cnt_sp.at[pl.ds(s * 5120, 2560)])
    pltpu.sync_copy(zb, cnt_sp.at[pl.ds(s * 5120 + 2560, 2560)])
    plsc.subcore_barrier()

    def count_chunk(ci, _):
      coff = s * EPC + ci * CH
      pltpu.sync_copy(dst_hbm.at[pl.ds(coff, CH)], dstc)
      pltpu.sync_copy(et_hbm.at[pl.ds(coff, CH)], tc)
      def count_body(bi, _):
        for kk in range(B // L):
          dv = dstc[pl.ds(bi * B + kk * L, L)]
          tv = tc[pl.ds(bi * B + kk * L, L)]
          idxb[pl.ds(kk * L, L)] = dv * 8 + tv
        pltpu.sync_copy(ones, cnt_sp.at[idxb], add=True)
        return 0
      lax.fori_loop(0, CH // B, count_body, 0)
      return 0
    lax.fori_loop(0, EPC // CH, count_chunk, 0)
    plsc.subcore_barrier()

    pltpu.sync_copy(cnt_sp.at[pl.ds(0, CNT_PAD - 1920)], invb)
    def inv_body(i, _):
      v = invb[pl.ds(i * L, L)]
      invb[pl.ds(i * L, L)] = 1.0 / jnp.maximum(v, 1.0)
      return 0
    lax.fori_loop(0, (CNT_PAD - 1920) // L, inv_body, 0)

    def w_chunk(ci, _):
      coff = wid * EPW + ci * CH
      pltpu.sync_copy(src_hbm.at[pl.ds(coff, CH)], srcc)
      pltpu.sync_copy(dst_hbm.at[pl.ds(coff, CH)], dstc)
      pltpu.sync_copy(et_hbm.at[pl.ds(coff, CH)], tc)
      def gi_body(i, _):
        sv = srcc[pl.ds(i * L, L)]
        tv = tc[pl.ds(i * L, L)]
        gc[pl.ds(i * L, L)] = tv * N + sv
        return 0
      lax.fori_loop(0, CH // L, gi_body, 0)
      pltpu.sync_copy(gc, gidx_hbm.at[pl.ds(coff, CH)])
      def w_body(bi, _):
        for kk in range(B // L):
          dv = dstc[pl.ds(bi * B + kk * L, L)]
          tv = tc[pl.ds(bi * B + kk * L, L)]
          wv = plsc.load_gather(invb, [dv * 8 + tv])
          for j in range(L):
            plsc.store_scatter(wb16, [kk * (L * L) + iot * L + j], wv)
        pltpu.sync_copy(
            wb16, w16_hbm.at[pl.ds((coff + bi * B) * L, B * L)])
        return 0
      lax.fori_loop(0, CH // B, w_body, 0)
      return 0
    lax.fori_loop(0, EPW // CH, w_chunk, 0)

  return k(src, dst, edge_type)


def _sc_layer(tabA, tabB, dst, gidx, w16):
  NBAT = EPC // B

  @functools.partial(
      pl.kernel,
      out_type=jax.ShapeDtypeStruct((2 * N, HF), _f32),
      mesh=_mesh(),
      compiler_params=pltpu.CompilerParams(needs_layout_passes=False),
      scratch_types=[
          pltpu.VMEM_SHARED((N, HF), _f32),
          pltpu.VMEM((16, HF), _f32),
          [pltpu.VMEM((B,), _i32) for _ in range(3)],
          [pltpu.VMEM((B,), _i32) for _ in range(3)],
          [pltpu.VMEM((B,), _i32) for _ in range(3)],
          [pltpu.VMEM((B * L,), _f32) for _ in range(3)],
          [pltpu.VMEM((B, HF), _f32) for _ in range(3)],
          pltpu.VMEM((16, HF), _f32),
          [pltpu.SemaphoreType.DMA for _ in range(3)],
          [pltpu.SemaphoreType.DMA for _ in range(3)],
          [pltpu.SemaphoreType.DMA for _ in range(3)],
          pltpu.SemaphoreType.DMA,
      ],
  )
  def k(tabA_hbm, tabB_hbm, dst_hbm, gidx_hbm, w16_hbm, out_hbm,
        acc_sp, zb, gidxb, dstS, dstX, wb16, rows, ob,
        semG, semI, semC, semZ):
    c = lax.axis_index("c")
    s = lax.axis_index("s")

    for i in range(16):
      for j in range(HF // L):
        zb[i, pl.ds(j * L, L)] = jnp.zeros((L,), _f32)
    def z_body(i, _):
      blk = s + i * NS
      @pl.when(blk < N // 16)
      def _():
        pltpu.async_copy(zb, acc_sp.at[pl.ds(blk * 16, 16)], semZ)
      return 0
    lax.fori_loop(0, (N // 16 + NS - 1) // NS, z_body, 0)
    def z_drain(i, _):
      blk = s + i * NS
      @pl.when(blk < N // 16)
      def _():
        pltpu.make_async_copy(zb, acc_sp.at[pl.ds(blk * 16, 16)],
                              semZ).wait()
      return 0
    lax.fori_loop(0, (N // 16 + NS - 1) // NS, z_drain, 0)
    plsc.subcore_barrier()

    def start_idx(slot, b):
      off = s * EPC + b * B
      pltpu.async_copy(dst_hbm.at[pl.ds(off, B)], dstS[slot], semI[slot])
      pltpu.async_copy(gidx_hbm.at[pl.ds(off, B)], gidxb[slot], semI[slot])

    def wait_idx(slot):
      pltpu.make_async_copy(dst_hbm.at[pl.ds(0, B)], dstS[slot],
                            semI[slot]).wait()
      pltpu.make_async_copy(gidx_hbm.at[pl.ds(0, B)], gidxb[slot],
                            semI[slot]).wait()

    def start_gw(slot, b):
      @pl.when(c == 0)
      def _():
        pltpu.async_copy(tabA_hbm.at[gidxb[slot]], rows[slot], semG[slot])
      @pl.when(c == 1)
      def _():
        pltpu.async_copy(tabB_hbm.at[gidxb[slot]], rows[slot], semG[slot])
      pltpu.async_copy(
          w16_hbm.at[pl.ds((s * EPC + b * B) * L, B * L)], wb16[slot],
          semG[slot])

    def wait_gw(slot):
      pltpu.make_async_copy(tabA_hbm.at[gidxb[slot]], rows[slot],
                            semG[slot]).wait()
      pltpu.make_async_copy(
          w16_hbm.at[pl.ds(0, B * L)], wb16[slot], semG[slot]).wait()

    def scale(slot):
      for e in range(B):
        wv = wb16[slot][pl.ds(e * L, L)]
        for j in range(HF // L):
          rows[slot][e, pl.ds(j * L, L)] = (
              rows[slot][e, pl.ds(j * L, L)] * wv)

    def start_scatter(slot):
      for kk in range(B // L):
        dstX[slot][pl.ds(kk * L, L)] = dstS[slot][pl.ds(kk * L, L)]
      pltpu.async_copy(rows[slot], acc_sp.at[dstX[slot]], semC[slot],
                       add=True)

    def wait_c(slot):
      pltpu.make_async_copy(tabA_hbm.at[gidxb[slot]], rows[slot],
                            semC[slot]).wait()

    for b in range(3):
      start_idx(b, b)
    for b in range(2):
      wait_idx(b)
      start_gw(b, b)

    def steady(p, _):
      b0 = p * 3
      for h in range(3):
        q, q2 = h, (h + 2) % 3
        wait_gw(q)
        scale(q)
        start_scatter(q)
        if h == 0:
          @pl.when(p > 0)
          def _():
            wait_c(q2)
        else:
          wait_c(q2)
        wait_idx(q2)
        start_gw(q2, b0 + h + 2)
        start_idx(q, b0 + h + 3)
      return 0
    lax.fori_loop(0, (NBAT - 4) // 3, steady, 0)

    for b in range(NBAT - 4, NBAT):
      q = b % 3
      wait_gw(q)
      scale(q)
      start_scatter(q)
      if b + 2 < NBAT:
        q2 = (b + 2) % 3
        wait_c(q2)
        wait_idx(q2)
        start_gw(q2, b + 2)
      if b + 3 < NBAT:
        start_idx(q, b + 3)
    wait_c((NBAT - 3) % 3)
    wait_c((NBAT - 2) % 3)
    wait_c((NBAT - 1) % 3)
    plsc.subcore_barrier()

    def wb_body(i, _):
      blk = s + i * NS
      @pl.when(blk < N // B)
      def _():
        pltpu.sync_copy(acc_sp.at[pl.ds(blk * B, B)], rows[0])
        pltpu.sync_copy(rows[0], out_hbm.at[pl.ds(c * N + blk * B, B)])
      return 0
    lax.fori_loop(0, (N // B + NS - 1) // NS, wb_body, 0)

  return k(tabA, tabB, dst, gidx, w16)


def _sc_pool(h3p, batch_p):
  NPAD = 10240
  NPW = NPAD // NW

  @functools.partial(
      pl.kernel,
      out_type=jax.ShapeDtypeStruct((NW * G * NHID,), _f32),
      mesh=_mesh(),
      compiler_params=pltpu.CompilerParams(needs_layout_passes=False),
      scratch_types=[
          pltpu.VMEM((G * NHID,), _f32),
          pltpu.VMEM((L, NHID), _f32),
          pltpu.VMEM((L,), _i32),
      ],
  )
  def k(h_hbm, b_hbm, out_hbm, gacc, rowsb, batchb):
    c = lax.axis_index("c")
    s = lax.axis_index("s")
    wid = s * NC + c

    def z_body(i, _):
      gacc[pl.ds(i * L, L)] = jnp.zeros((L,), _f32)
      return 0
    lax.fori_loop(0, G * NHID // L, z_body, 0)

    iot = _iota16()

    def chunk_body(kk, _):
      off = wid * NPW + kk * L
      @pl.when(off < N)
      def _():
        pltpu.sync_copy(h_hbm.at[pl.ds(off, L)], rowsb)
        pltpu.sync_copy(b_hbm.at[pl.ds(off, L)], batchb)
        bv = batchb[pl.ds(0, L)]
        for m in range(L):
          gid = lax.reduce_max(jnp.where(iot == m, bv, 0), axes=(0,))
          base = gid * NHID
          for j in range(NHID // L):
            idxv = base + (j * L + iot)
            cur = plsc.load_gather(gacc, [idxv])
            nv = jnp.maximum(cur, rowsb[m, pl.ds(j * L, L)])
            plsc.store_scatter(gacc, [idxv], nv)
      return 0
    lax.fori_loop(0, NPW // L, chunk_body, 0)

    pltpu.sync_copy(gacc, out_hbm.at[pl.ds(wid * (G * NHID), G * NHID)])

  return k(h3p, batch_p)


BN = 400
NB = N // BN


def _tc_head(x, W, root, b):
  def body(x_ref, w_ref, root_ref, b_ref, h2a_ref, h2b_ref, r1_ref):
    r = pl.program_id(1)
    xb = x_ref[...]
    h = jnp.dot(xb, w_ref[0], preferred_element_type=_f32)
    h2a_ref[0] = h[:, :HF]
    h2b_ref[0] = h[:, HF:]
    @pl.when(r == 0)
    def _():
      r1_ref[...] = (jnp.dot(xb, root_ref[...], preferred_element_type=_f32)
                     + b_ref[...])

  d = x.shape[1]
  return pl.pallas_call(
      body,
      grid=(NB, R),
      in_specs=[
          pl.BlockSpec((BN, d), lambda i, r: (i, 0)),
          pl.BlockSpec((1, d, NHID), lambda i, r: (r, 0, 0)),
          pl.BlockSpec((d, NHID), lambda i, r: (0, 0)),
          pl.BlockSpec((1, NHID), lambda i, r: (0, 0)),
      ],
      out_specs=[
          pl.BlockSpec((1, BN, HF), lambda i, r: (r, i, 0)),
          pl.BlockSpec((1, BN, HF), lambda i, r: (r, i, 0)),
          pl.BlockSpec((BN, NHID), lambda i, r: (i, 0)),
      ],
      out_shape=[
          jax.ShapeDtypeStruct((R, N, HF), _f32),
          jax.ShapeDtypeStruct((R, N, HF), _f32),
          jax.ShapeDtypeStruct((N, NHID), _f32),
      ],
  )(x, W, root, b.reshape(1, NHID))


def _tc_mid(Rprev, msg, W, root, b):
  def body(rp_ref, m_ref, w_ref, root_ref, b_ref, h2a_ref, h2b_ref, rn_ref):
    r = pl.program_id(1)
    h = jax.nn.relu(rp_ref[...] +
                    jnp.concatenate([m_ref[0], m_ref[1]], axis=1))
    hh = jnp.dot(h, w_ref[0], preferred_element_type=_f32)
    h2a_ref[0] = hh[:, :HF]
    h2b_ref[0] = hh[:, HF:]
    @pl.when(r == 0)
    def _():
      rn_ref[...] = (jnp.dot(h, root_ref[...], preferred_element_type=_f32)
                     + b_ref[...])

  return pl.pallas_call(
      body,
      grid=(NB, R),
      in_specs=[
          pl.BlockSpec((BN, NHID), lambda i, r: (i, 0)),
          pl.BlockSpec((2, BN, HF), lambda i, r: (0, i, 0)),
          pl.BlockSpec((1, NHID, NHID), lambda i, r: (r, 0, 0)),
          pl.BlockSpec((NHID, NHID), lambda i, r: (0, 0)),
          pl.BlockSpec((1, NHID), lambda i, r: (0, 0)),
      ],
      out_specs=[
          pl.BlockSpec((1, BN, HF), lambda i, r: (r, i, 0)),
          pl.BlockSpec((1, BN, HF), lambda i, r: (r, i, 0)),
          pl.BlockSpec((BN, NHID), lambda i, r: (i, 0)),
      ],
      out_shape=[
          jax.ShapeDtypeStruct((R, N, HF), _f32),
          jax.ShapeDtypeStruct((R, N, HF), _f32),
          jax.ShapeDtypeStruct((N, NHID), _f32),
      ],
  )(Rprev, msg, W, root, b.reshape(1, NHID))


def _tc_relu(Rprev, msg):
  def body(rp_ref, m_ref, o_ref):
    o_ref[...] = jax.nn.relu(rp_ref[...] +
                             jnp.concatenate([m_ref[0], m_ref[1]], axis=1))

  return pl.pallas_call(
      body,
      grid=(NB,),
      in_specs=[
          pl.BlockSpec((BN, NHID), lambda i: (i, 0)),
          pl.BlockSpec((2, BN, HF), lambda i: (0, i, 0)),
      ],
      out_specs=pl.BlockSpec((BN, NHID), lambda i: (i, 0)),
      out_shape=jax.ShapeDtypeStruct((N, NHID), _f32),
  )(Rprev, msg)


def _tc_pool_mlp(parts, Wm1, bm1, Wm2, bm2):
  def body(p_ref, w1_ref, b1_ref, w2_ref, b2_ref, o_ref):
    g = jnp.max(p_ref[...], axis=0)
    gg = jax.nn.relu(jnp.dot(g, w1_ref[...], preferred_element_type=_f32)
                     + b1_ref[...])
    o_ref[...] = (jnp.dot(gg, w2_ref[...], preferred_element_type=_f32)
                  + b2_ref[...])

  return pl.pallas_call(
      body,
      out_shape=jax.ShapeDtypeStruct((G, NOUT), _f32),
  )(parts, Wm1, bm1.reshape(1, NHID), Wm2, bm2.reshape(1, NOUT))


def kernel(x, edge_index, edge_type, batch,
           W1, root1, b1, W2, root2, b2, W3, root3, b3,
           Wm1, bm1, Wm2, bm2):
  src = edge_index[0]
  dst = edge_index[1]
  w16, gidx = _sc_weights(src, dst, edge_type)

  Ha, Hb, R1 = _tc_head(x, W1, root1, b1)
  msg1 = _sc_layer(Ha.reshape(R * N, HF), Hb.reshape(R * N, HF),
                   dst, gidx, w16)

  Ha, Hb, R2 = _tc_mid(R1, msg1.reshape(2, N, HF), W2, root2, b2)
  msg2 = _sc_layer(Ha.reshape(R * N, HF), Hb.reshape(R * N, HF),
                   dst, gidx, w16)

  Ha, Hb, R3 = _tc_mid(R2, msg2.reshape(2, N, HF), W3, root3, b3)
  msg3 = _sc_layer(Ha.reshape(R * N, HF), Hb.reshape(R * N, HF),
                   dst, gidx, w16)

  h3 = _tc_relu(R3, msg3.reshape(2, N, HF))

  parts = _sc_pool(h3, batch)

  return _tc_pool_mlp(parts.reshape(NW, G, NHID), Wm1, bm1, Wm2, bm2)
